# baseline (speedup 1.0000x reference)
import functools

import jax
import jax.numpy as jnp
from jax.experimental import pallas as pl
from jax.experimental.pallas import tpu as pltpu

_NEG_INF = -1e30


def _mha_kernel(H, d, tq, nq, nb, q_ref, k_ref, v_ref, wq_ref, bq_ref,
                wk_ref, bk_ref, wv_ref, bv_ref, wo_ref, bo_ref, out_ref,
                k_sc, v_sc):
    rows = jax.lax.broadcasted_iota(jnp.int32, (tq, tq), 0)
    cols = jax.lax.broadcasted_iota(jnp.int32, (tq, tq), 1)
    tri = jnp.where(rows >= cols, 0.0, _NEG_INF)

    for bb in range(nb):
        kx = k_ref[bb].astype(jnp.bfloat16)
        k_all = jnp.dot(kx, wk_ref[...],
                        preferred_element_type=jnp.float32) + bk_ref[...]
        k_sc[bb] = k_all.astype(jnp.bfloat16)
        vx = v_ref[bb].astype(jnp.bfloat16)
        v_all = jnp.dot(vx, wv_ref[...],
                        preferred_element_type=jnp.float32) + bv_ref[...]
        v_sc[bb] = v_all.astype(jnp.bfloat16)

        x = q_ref[bb].astype(jnp.bfloat16)
        q_all = jnp.dot(x, wq_ref[...],
                        preferred_element_type=jnp.float32) + bq_ref[...]

        for qi in range(nq):
            ctxs = []
            for h in range(H):
                q_h = q_all[qi * tq:(qi + 1) * tq,
                            h * d:(h + 1) * d].astype(jnp.bfloat16)
                r = None
                for j in range(qi + 1):
                    s = jax.lax.dot_general(
                        q_h,
                        k_sc[bb, j * tq:(j + 1) * tq, h * d:(h + 1) * d],
                        (((1,), (1,)), ((), ())),
                        preferred_element_type=jnp.float32)
                    if j == qi:
                        s = s + tri
                    p = jnp.exp2(s).astype(jnp.bfloat16)
                    rj = jnp.dot(
                        p,
                        v_sc[bb, j * tq:(j + 1) * tq,
                             h * 2 * d:(h + 1) * 2 * d],
                        preferred_element_type=jnp.float32)
                    r = rj if r is None else r + rj
                ctxs.append((r[:, 0:d] / r[:, d:2 * d]).astype(jnp.bfloat16))
            cat = jnp.concatenate(ctxs, axis=1)
            out = jnp.dot(cat, wo_ref[...],
                          preferred_element_type=jnp.float32)
            out_ref[bb, qi * tq:(qi + 1) * tq] = (out + bo_ref[...]).astype(
                out_ref.dtype)


def kernel(query, key, value, wq, bq, wk, bk, wv, bv, wo, bo):
    B, S, D = query.shape
    H, _, dq = wq.shape
    d = wk.shape[-1]
    assert dq == d
    bf = jnp.bfloat16
    f32 = jnp.float32

    inv = float(dq) ** -0.5 * 1.4426950408889634
    wq_c = jnp.transpose(wq * inv, (1, 0, 2)).reshape(D, H * d).astype(bf)
    bq_c = (bq * inv).reshape(1, H * d).astype(f32)
    wk_c = jnp.transpose(wk, (1, 0, 2)).reshape(D, H * d).astype(bf)
    bk_c = bk.reshape(1, H * d).astype(f32)
    wv_t = jnp.transpose(wv, (1, 0, 2))
    wv_c = jnp.concatenate(
        [wv_t, jnp.zeros_like(wv_t)], axis=2).reshape(D, H * 2 * d).astype(bf)
    bv_c = jnp.concatenate(
        [bv, jnp.ones_like(bv)], axis=1).reshape(1, H * 2 * d).astype(f32)
    wo2 = wo.astype(bf)
    bo2 = bo.astype(f32)

    tq = 256 if S % 256 == 0 else S
    nq = S // tq
    nb = 2 if B % 2 == 0 else 1

    kern = functools.partial(_mha_kernel, H, d, tq, nq, nb)
    return pl.pallas_call(
        kern,
        out_shape=jax.ShapeDtypeStruct((B, S, D), query.dtype),
        grid=(B // nb,),
        in_specs=[
            pl.BlockSpec((nb, S, D), lambda b: (b, 0, 0)),
            pl.BlockSpec((nb, S, D), lambda b: (b, 0, 0)),
            pl.BlockSpec((nb, S, D), lambda b: (b, 0, 0)),
            pl.BlockSpec((D, H * d), lambda b: (0, 0)),
            pl.BlockSpec((1, H * d), lambda b: (0, 0)),
            pl.BlockSpec((D, H * d), lambda b: (0, 0)),
            pl.BlockSpec((1, H * d), lambda b: (0, 0)),
            pl.BlockSpec((D, H * 2 * d), lambda b: (0, 0)),
            pl.BlockSpec((1, H * 2 * d), lambda b: (0, 0)),
            pl.BlockSpec((H * d, D), lambda b: (0, 0)),
            pl.BlockSpec((1, D), lambda b: (0, 0)),
        ],
        out_specs=pl.BlockSpec((nb, S, D), lambda b: (b, 0, 0)),
        scratch_shapes=[
            pltpu.VMEM((nb, S, H * d), bf),
            pltpu.VMEM((nb, S, H * 2 * d), bf),
        ],
        compiler_params=pltpu.CompilerParams(
            dimension_semantics=("parallel",)),
    )(query, key, value, wq_c, bq_c, wk_c, bk_c, wv_c, bv_c, wo2, bo2)


# speedup vs baseline: 1.0397x; 1.0397x over previous
import functools

import jax
import jax.numpy as jnp
from jax.experimental import pallas as pl
from jax.experimental.pallas import tpu as pltpu

_NEG_INF = -1e30


def _mha_kernel(H, d, tq, nq, nb, q_ref, k_ref, v_ref, wq_ref, bq_ref,
                wk_ref, bk_ref, wv_ref, bv_ref, wo_ref, bo_ref, out_ref,
                k_sc, v_sc):
    S = k_ref.shape[1]

    rows = jax.lax.broadcasted_iota(jnp.int32, (tq, tq), 0)
    cols = jax.lax.broadcasted_iota(jnp.int32, (tq, tq), 1)
    tri = jnp.where(rows >= cols, 0.0, _NEG_INF)
    ones_cols = jnp.ones((S, d), jnp.bfloat16)

    for bb in range(nb):
        kx = k_ref[bb].astype(jnp.bfloat16)
        k_all = jnp.dot(kx, wk_ref[...],
                        preferred_element_type=jnp.float32) + bk_ref[...]
        vx = v_ref[bb].astype(jnp.bfloat16)
        v_all = jnp.dot(vx, wv_ref[...],
                        preferred_element_type=jnp.float32) + bv_ref[...]
        for h in range(H):
            k_sc[bb, h] = k_all[:, h * d:(h + 1) * d].astype(jnp.bfloat16)
            v_sc[bb, h, :, 0:d] = v_all[:, h * d:(h + 1) * d].astype(
                jnp.bfloat16)
            v_sc[bb, h, :, d:2 * d] = ones_cols

        x = q_ref[bb].astype(jnp.bfloat16)
        q_all = jnp.dot(x, wq_ref[...],
                        preferred_element_type=jnp.float32) + bq_ref[...]

        for qi in range(nq):
            ctxs = []
            for h in range(H):
                q_h = q_all[qi * tq:(qi + 1) * tq,
                            h * d:(h + 1) * d].astype(jnp.bfloat16)
                r = None
                for j in range(qi + 1):
                    s = jax.lax.dot_general(
                        q_h, k_sc[bb, h, j * tq:(j + 1) * tq],
                        (((1,), (1,)), ((), ())),
                        preferred_element_type=jnp.float32)
                    if j == qi:
                        s = s + tri
                    p = jnp.exp2(s).astype(jnp.bfloat16)
                    rj = jnp.dot(p, v_sc[bb, h, j * tq:(j + 1) * tq],
                                 preferred_element_type=jnp.float32)
                    r = rj if r is None else r + rj
                ctxs.append((r[:, 0:d] *
                             pl.reciprocal(r[:, d:2 * d], approx=True)
                             ).astype(jnp.bfloat16))
            cat = jnp.concatenate(ctxs, axis=1)
            out = jnp.dot(cat, wo_ref[...],
                          preferred_element_type=jnp.float32)
            out_ref[bb, qi * tq:(qi + 1) * tq] = (out + bo_ref[...]).astype(
                out_ref.dtype)


def kernel(query, key, value, wq, bq, wk, bk, wv, bv, wo, bo):
    B, S, D = query.shape
    H, _, dq = wq.shape
    d = wk.shape[-1]
    assert dq == d
    bf = jnp.bfloat16
    f32 = jnp.float32

    inv = float(dq) ** -0.5 * 1.4426950408889634
    wq_c = jnp.transpose(wq * inv, (1, 0, 2)).reshape(D, H * d).astype(bf)
    bq_c = (bq * inv).reshape(1, H * d).astype(f32)
    wk_c = jnp.transpose(wk, (1, 0, 2)).reshape(D, H * d).astype(bf)
    bk_c = bk.reshape(1, H * d).astype(f32)
    wv_c = jnp.transpose(wv, (1, 0, 2)).reshape(D, H * d).astype(bf)
    bv_c = bv.reshape(1, H * d).astype(f32)
    wo2 = wo.astype(bf)
    bo2 = bo.astype(f32)

    tq = 256 if S % 256 == 0 else S
    nq = S // tq
    nb = 2 if B % 2 == 0 else 1

    kern = functools.partial(_mha_kernel, H, d, tq, nq, nb)
    return pl.pallas_call(
        kern,
        out_shape=jax.ShapeDtypeStruct((B, S, D), query.dtype),
        grid=(B // nb,),
        in_specs=[
            pl.BlockSpec((nb, S, D), lambda b: (b, 0, 0)),
            pl.BlockSpec((nb, S, D), lambda b: (b, 0, 0)),
            pl.BlockSpec((nb, S, D), lambda b: (b, 0, 0)),
            pl.BlockSpec((D, H * d), lambda b: (0, 0)),
            pl.BlockSpec((1, H * d), lambda b: (0, 0)),
            pl.BlockSpec((D, H * d), lambda b: (0, 0)),
            pl.BlockSpec((1, H * d), lambda b: (0, 0)),
            pl.BlockSpec((D, H * d), lambda b: (0, 0)),
            pl.BlockSpec((1, H * d), lambda b: (0, 0)),
            pl.BlockSpec((H * d, D), lambda b: (0, 0)),
            pl.BlockSpec((1, D), lambda b: (0, 0)),
        ],
        out_specs=pl.BlockSpec((nb, S, D), lambda b: (b, 0, 0)),
        scratch_shapes=[
            pltpu.VMEM((nb, H, S, d), bf),
            pltpu.VMEM((nb, H, S, 2 * d), bf),
        ],
        compiler_params=pltpu.CompilerParams(
            dimension_semantics=("parallel",)),
    )(query, key, value, wq_c, bq_c, wk_c, bk_c, wv_c, bv_c, wo2, bo2)
